# baseline (speedup 1.0000x reference)
import functools

import jax
import jax.numpy as jnp
from jax import lax
from jax.experimental import pallas as pl
from jax.experimental.pallas import tpu as pltpu
from jax.experimental.pallas import tpu_sc as plsc

N = 10000
D = 128
NC = 2
NS = 16
NT = NC * NS
CH = 64
CPT = 160
HCPT = CPT // 2
EPT = CPT * CH
EP = NT * EPT
NPAD = 10240
RPS = NPAD // NS

_mesh = plsc.VectorSubcoreMesh(core_axis_name="c", subcore_axis_name="s")
_sc_params = pltpu.CompilerParams(needs_layout_passes=False)


@functools.partial(
    pl.kernel,
    mesh=_mesh,
    out_type=jax.ShapeDtypeStruct((NT, NPAD), jnp.float32),
    compiler_params=_sc_params,
    scratch_types=[
        pltpu.VMEM((EPT,), jnp.int32),
        pltpu.VMEM((NPAD,), jnp.float32),
    ],
)
def _deg_kernel(dst_hbm, out_hbm, dst_v, hist_v):
    c = lax.axis_index("c")
    s = lax.axis_index("s")
    t = c * NS + s
    pltpu.sync_copy(dst_hbm.at[t], dst_v)

    def zero_body(i, carry):
        hist_v[pl.ds(i * 16, 16)] = jnp.zeros((16,), jnp.float32)
        return carry

    lax.fori_loop(0, NPAD // 16, zero_body, 0)

    ones = jnp.ones((16,), jnp.float32)

    def body(i, carry):
        idx = dst_v[pl.ds(i * 16, 16)]
        plsc.addupdate_scatter(hist_v, [idx], ones)
        return carry

    lax.fori_loop(0, EPT // 16, body, 0)
    pltpu.sync_copy(hist_v, out_hbm.at[t])


@functools.partial(
    pl.kernel,
    mesh=_mesh,
    out_type=jax.ShapeDtypeStruct((NC, NPAD, D), jnp.float32),
    compiler_params=_sc_params,
    scratch_types=[
        pltpu.VMEM((HCPT, 2 * CH), jnp.int32),
        pltpu.VMEM((CH, D), jnp.float32),
        pltpu.VMEM((CH, D), jnp.float32),
        pltpu.VMEM((CH, D), jnp.float32),
        pltpu.VMEM((CH, D), jnp.float32),
        pltpu.VMEM_SHARED((NPAD, D), jnp.float32),
        pltpu.SemaphoreType.DMA,
        pltpu.SemaphoreType.DMA,
        pltpu.SemaphoreType.DMA,
        pltpu.SemaphoreType.DMA,
        pltpu.SemaphoreType.DMA,
        pltpu.SemaphoreType.DMA,
        pltpu.SemaphoreType.DMA,
        pltpu.SemaphoreType.DMA,
    ],
)
def _edge_kernel(
    h_hbm, eidx_hbm, out_hbm,
    idx_v, r0, r1, r2, r3, acc,
    g0s, g1s, g2s, g3s, s0s, s1s, s2s, s3s,
):
    c = lax.axis_index("c")
    s = lax.axis_index("s")
    t = c * NS + s
    rows = [r0, r1, r2, r3]
    gsem = [g0s, g1s, g2s, g3s]
    ssem = [s0s, s1s, s2s, s3s]

    def zbody(i, carry):
        r = i // 8
        j = i % 8
        r0[r, pl.ds(j * 16, 16)] = jnp.zeros((16,), jnp.float32)
        return carry

    lax.fori_loop(0, CH * 8, zbody, 0)

    def zcopy(k, carry):
        pltpu.sync_copy(r0, acc.at[pl.ds(s * RPS + k * CH, CH)])
        return carry

    lax.fori_loop(0, RPS // CH, zcopy, 0)
    plsc.subcore_barrier()

    def gstart(g, b):
        pltpu.async_copy(h_hbm.at[idx_v.at[g, pl.ds(0, CH)]], rows[b], gsem[b])

    def gwait(g, b):
        pltpu.make_async_copy(
            h_hbm.at[idx_v.at[g, pl.ds(0, CH)]], rows[b], gsem[b]
        ).wait()

    def sstart(g, b):
        for i in range(CH // 16):
            vec = idx_v[g, pl.ds(CH + i * 16, 16)]
            pltpu.async_copy(
                rows[b].at[pl.ds(i * 16, 16)], acc.at[vec], ssem[b], add=True
            )

    def swait(g, b):
        for i in range(CH // 16):
            vec = idx_v[g, pl.ds(CH + i * 16, 16)]
            pltpu.make_async_copy(
                rows[b].at[pl.ds(i * 16, 16)], acc.at[vec], ssem[b]
            ).wait()

    for half in range(2):
        pltpu.sync_copy(eidx_hbm.at[t, pl.ds(half * HCPT, HCPT)], idx_v)
        gstart(0, 0)
        gstart(1, 1)

        def step(k, carry):
            for b in range(4):
                g = 4 * k + b
                gwait(g, b)
                sstart(g, b)
                bw = (b + 2) % 4
                gw = jnp.maximum(g - 2, 0)
                if b < 2:
                    @pl.when(k > 0)
                    def _():
                        swait(gw, bw)
                    gstart(g + 2, bw)
                else:
                    swait(gw, bw)

                    @pl.when(k < HCPT // 4 - 1)
                    def _():
                        gstart(g + 2, bw)
            return carry

        lax.fori_loop(0, HCPT // 4, step, 0)
        swait(HCPT - 2, 2)
        swait(HCPT - 1, 3)
    plsc.subcore_barrier()

    def ocopy(k, carry):
        pltpu.sync_copy(
            acc.at[pl.ds(s * RPS + k * CH, CH)],
            out_hbm.at[c, pl.ds(s * RPS + k * CH, CH)],
        )
        return carry

    lax.fori_loop(0, RPS // CH, ocopy, 0)


def _m0_body(x_ref, w_ref, m_ref):
    m_ref[...] = jnp.dot(x_ref[...], w_ref[...], preferred_element_type=jnp.float32)


def _m0(x, w):
    return pl.pallas_call(
        _m0_body,
        out_shape=jax.ShapeDtypeStruct((N, D), jnp.float32),
    )(x, w)


def _scale_body(hists_ref, m_ref, h_ref, dis_ref):
    deg = jnp.sum(hists_ref[:, :N], axis=0) + 1.0
    dis = lax.rsqrt(deg)[:, None]
    dis_ref[...] = dis
    h_ref[...] = m_ref[...] * dis


def _scale(hists, m):
    return pl.pallas_call(
        _scale_body,
        out_shape=(
            jax.ShapeDtypeStruct((N, D), jnp.float32),
            jax.ShapeDtypeStruct((N, 1), jnp.float32),
        ),
    )(hists, m)


def _fuse_body(p_ref, h_ref, dis_ref, b_ref, w_ref, o_ref):
    dis = dis_ref[...]
    a = p_ref[0, :N] + p_ref[1, :N] + h_ref[...]
    x = jnp.maximum(a * dis + b_ref[...], 0.0)
    o_ref[...] = jnp.dot(x * dis, w_ref[...], preferred_element_type=jnp.float32)


def _fuse(p, h, dis, b, w):
    return pl.pallas_call(
        _fuse_body,
        out_shape=jax.ShapeDtypeStruct((N, D), jnp.float32),
    )(p, h, dis, b, w)


def _final_body(p_ref, h_ref, dis_ref, b_ref, o_ref):
    a = p_ref[0, :N] + p_ref[1, :N] + h_ref[...]
    o_ref[...] = a * dis_ref[...] + b_ref[...]


def _final(p, h, dis, b):
    return pl.pallas_call(
        _final_body,
        out_shape=jax.ShapeDtypeStruct((N, D), jnp.float32),
    )(p, h, dis, b)


def kernel(x, edge_index, W0, b0, W1, b1, W2, b2, W3, b3):
    src = edge_index[0].astype(jnp.int32)
    dst = edge_index[1].astype(jnp.int32)
    e = src.shape[0]
    pad_n = EP - e
    ar = jnp.arange(pad_n, dtype=jnp.int32)
    src_p = jnp.concatenate([src, ar % N]).reshape(NT, CPT, CH)
    dst_p = jnp.concatenate([dst, N + ar % (NPAD - N)]).reshape(NT, CPT, CH)
    eidx = jnp.concatenate([src_p, dst_p], axis=-1)
    dst_flat = dst_p.reshape(NT, EPT)

    m = _m0(x, W0)
    hists = _deg_kernel(dst_flat)
    h, dis = _scale(hists, m)
    b_prev = [b0, b1, b2]
    w_next = [W1, W2, W3]
    for i in range(3):
        p = _edge_kernel(h, eidx)
        h = _fuse(p, h, dis, b_prev[i].reshape(1, D), w_next[i])
    p = _edge_kernel(h, eidx)
    return _final(p, h, dis, b3.reshape(1, D))


# speedup vs baseline: 1.0934x; 1.0934x over previous
import functools

import jax
import jax.numpy as jnp
from jax import lax
from jax.experimental import pallas as pl
from jax.experimental.pallas import tpu as pltpu
from jax.experimental.pallas import tpu_sc as plsc

N = 10000
D = 128
NC = 2
NS = 16
NT = NC * NS
CH = 128
CPT = 80
EPT = CPT * CH
EP = NT * EPT
NPAD = 10240
RPS = NPAD // NS

_mesh = plsc.VectorSubcoreMesh(core_axis_name="c", subcore_axis_name="s")
_sc_params = pltpu.CompilerParams(needs_layout_passes=False)


@functools.partial(
    pl.kernel,
    mesh=_mesh,
    out_type=jax.ShapeDtypeStruct((NT, NPAD), jnp.float32),
    compiler_params=_sc_params,
    scratch_types=[
        pltpu.VMEM((EPT,), jnp.int32),
        pltpu.VMEM((NPAD,), jnp.float32),
    ],
)
def _deg_kernel(dst_hbm, out_hbm, dst_v, hist_v):
    c = lax.axis_index("c")
    s = lax.axis_index("s")
    t = c * NS + s
    pltpu.sync_copy(dst_hbm.at[t], dst_v)

    def zero_body(i, carry):
        hist_v[pl.ds(i * 16, 16)] = jnp.zeros((16,), jnp.float32)
        return carry

    lax.fori_loop(0, NPAD // 16, zero_body, 0)

    ones = jnp.ones((16,), jnp.float32)

    def body(i, carry):
        idx = dst_v[pl.ds(i * 16, 16)]
        plsc.addupdate_scatter(hist_v, [idx], ones)
        return carry

    lax.fori_loop(0, EPT // 16, body, 0)
    pltpu.sync_copy(hist_v, out_hbm.at[t])


@functools.partial(
    pl.kernel,
    mesh=_mesh,
    out_type=jax.ShapeDtypeStruct((NC, NPAD, D), jnp.float32),
    compiler_params=_sc_params,
    scratch_types=[
        pltpu.VMEM((CPT // 2, CH), jnp.int32),
        pltpu.VMEM((CPT // 2, CH), jnp.int32),
        pltpu.VMEM((CH, D), jnp.float32),
        pltpu.VMEM((CH, D), jnp.float32),
        pltpu.VMEM_SHARED((NPAD, D), jnp.float32),
        pltpu.SemaphoreType.DMA,
        pltpu.SemaphoreType.DMA,
    ],
)
def _edge_kernel(
    h_hbm, src_hbm, dst_hbm, out_hbm, src_v, dst_v, rows_a, rows_b, acc, sem_a, sem_b
):
    c = lax.axis_index("c")
    s = lax.axis_index("s")
    t = c * NS + s
    hcpt = CPT // 2

    def zbody(i, carry):
        r = i // 8
        j = i % 8
        rows_a[r, pl.ds(j * 16, 16)] = jnp.zeros((16,), jnp.float32)
        return carry

    lax.fori_loop(0, CH * 8, zbody, 0)

    def zcopy(k, carry):
        pltpu.sync_copy(rows_a, acc.at[pl.ds(s * RPS + k * CH, CH)])
        return carry

    lax.fori_loop(0, RPS // CH, zcopy, 0)
    plsc.subcore_barrier()

    npair = hcpt // 2
    for half in range(2):
        pltpu.sync_copy(src_hbm.at[t, pl.ds(half * hcpt, hcpt)], src_v)
        pltpu.sync_copy(dst_hbm.at[t, pl.ds(half * hcpt, hcpt)], dst_v)
        pltpu.async_copy(h_hbm.at[src_v.at[0]], rows_a, sem_a)

        def body(k, carry):
            g0 = 2 * k
            g1 = g0 + 1
            pltpu.async_copy(h_hbm.at[src_v.at[g1]], rows_b, sem_b)
            pltpu.make_async_copy(h_hbm.at[src_v.at[g0]], rows_a, sem_a).wait()
            pltpu.sync_copy(rows_a, acc.at[dst_v.at[g0]], add=True)

            @pl.when(k < npair - 1)
            def _():
                pltpu.async_copy(h_hbm.at[src_v.at[g0 + 2]], rows_a, sem_a)

            pltpu.make_async_copy(h_hbm.at[src_v.at[g1]], rows_b, sem_b).wait()
            pltpu.sync_copy(rows_b, acc.at[dst_v.at[g1]], add=True)
            return carry

        lax.fori_loop(0, npair, body, 0)
    plsc.subcore_barrier()

    def ocopy(k, carry):
        pltpu.sync_copy(
            acc.at[pl.ds(s * RPS + k * CH, CH)],
            out_hbm.at[c, pl.ds(s * RPS + k * CH, CH)],
        )
        return carry

    lax.fori_loop(0, RPS // CH, ocopy, 0)


def _m0_body(x_ref, w_ref, m_ref):
    m_ref[...] = jnp.dot(x_ref[...], w_ref[...], preferred_element_type=jnp.float32)


def _m0(x, w):
    return pl.pallas_call(
        _m0_body,
        out_shape=jax.ShapeDtypeStruct((N, D), jnp.float32),
    )(x, w)


def _scale_body(hists_ref, m_ref, h_ref, dis_ref):
    deg = jnp.sum(hists_ref[:, :N], axis=0) + 1.0
    dis = lax.rsqrt(deg)[:, None]
    dis_ref[...] = dis
    h_ref[...] = m_ref[...] * dis


def _scale(hists, m):
    return pl.pallas_call(
        _scale_body,
        out_shape=(
            jax.ShapeDtypeStruct((N, D), jnp.float32),
            jax.ShapeDtypeStruct((N, 1), jnp.float32),
        ),
    )(hists, m)


def _fuse_body(p_ref, h_ref, dis_ref, b_ref, w_ref, o_ref):
    dis = dis_ref[...]
    a = p_ref[0, :N] + p_ref[1, :N] + h_ref[...]
    x = jnp.maximum(a * dis + b_ref[...], 0.0)
    o_ref[...] = jnp.dot(x * dis, w_ref[...], preferred_element_type=jnp.float32)


def _fuse(p, h, dis, b, w):
    return pl.pallas_call(
        _fuse_body,
        out_shape=jax.ShapeDtypeStruct((N, D), jnp.float32),
    )(p, h, dis, b, w)


def _final_body(p_ref, h_ref, dis_ref, b_ref, o_ref):
    a = p_ref[0, :N] + p_ref[1, :N] + h_ref[...]
    o_ref[...] = a * dis_ref[...] + b_ref[...]


def _final(p, h, dis, b):
    return pl.pallas_call(
        _final_body,
        out_shape=jax.ShapeDtypeStruct((N, D), jnp.float32),
    )(p, h, dis, b)


def kernel(x, edge_index, W0, b0, W1, b1, W2, b2, W3, b3):
    src = edge_index[0].astype(jnp.int32)
    dst = edge_index[1].astype(jnp.int32)
    e = src.shape[0]
    pad_n = EP - e
    ar = jnp.arange(pad_n, dtype=jnp.int32)
    src_p = jnp.concatenate([src, ar % N]).reshape(NT, CPT, CH)
    dst_p = jnp.concatenate([dst, N + ar % (NPAD - N)]).reshape(NT, CPT, CH)
    dst_flat = dst_p.reshape(NT, EPT)

    m = _m0(x, W0)
    hists = _deg_kernel(dst_flat)
    h, dis = _scale(hists, m)
    b_prev = [b0, b1, b2]
    w_next = [W1, W2, W3]
    for i in range(3):
        p = _edge_kernel(h, src_p, dst_p)
        h = _fuse(p, h, dis, b_prev[i].reshape(1, D), w_next[i])
    p = _edge_kernel(h, src_p, dst_p)
    return _final(p, h, dis, b3.reshape(1, D))
